# baseline (speedup 1.0000x reference)
import functools

import jax
import jax.numpy as jnp
from jax import lax
from jax.experimental import pallas as pl
from jax.experimental.pallas import tpu as pltpu
from jax.experimental.pallas import tpu_sc as plsc

B = 16384
D = 32
NC, NS, L = 2, 16, 16
NW = NC * NS
BPW = B // NW
CHUNK = 128
NCH = BPW // CHUNK

MODULUS_WEIGHT = 4.0
PHASE_WEIGHT = 1.0

_INV_2PI = float(0.15915494309189535)
_MAGIC = float(12582912.0)
_PI_HI = float(3.1415927410125732)
_PI_LO = float(-8.742277657347586e-08)
_C3 = float(-1.0 / 6.0)
_C5 = float(1.0 / 120.0)
_C7 = float(-1.0 / 5040.0)
_C9 = float(1.0 / 362880.0)
_C11 = float(-1.0 / 39916800.0)


def _abs_sin_half(p):
    z = p * _INV_2PI
    k = (z + _MAGIC) - _MAGIC
    y = p * 0.5 - k * _PI_HI
    y = y - k * _PI_LO
    y2 = y * y
    t = _C9 + y2 * _C11
    t = _C7 + y2 * t
    t = _C5 + y2 * t
    t = _C3 + y2 * t
    t = 1.0 + y2 * t
    return jnp.abs(y * t)


def _newton_sqrt(x):
    i = lax.bitcast_convert_type(x, jnp.int32)
    i = jnp.int32(0x1FBD1DF5) + lax.shift_right_arithmetic(i, 1)
    y = lax.bitcast_convert_type(i, jnp.float32)
    for _ in range(3):
        y = 0.5 * (y + x / y)
    return y


def _sc_body(head, relation, tail, emt, ept, r_mod, r_ph, sm, sp, out, *scr):
    ih = scr[0:NCH]
    it = scr[NCH:2 * NCH]
    ir = scr[2 * NCH:3 * NCH]
    hm, hp, tm, tp = scr[3 * NCH:3 * NCH + 4]
    rm, rp = scr[3 * NCH + 4:3 * NCH + 6]
    ihr, itr = scr[3 * NCH + 6:3 * NCH + 8]
    smv, spv = scr[3 * NCH + 8:3 * NCH + 10]
    out_v = scr[3 * NCH + 10]
    sem = scr[3 * NCH + 11]

    wid = lax.axis_index("s") * NC + lax.axis_index("c")
    base = wid * BPW

    pltpu.sync_copy(head.at[pl.ds(base, BPW)], ihr)
    pltpu.sync_copy(tail.at[pl.ds(base, BPW)], itr)
    pltpu.sync_copy(sm, smv)
    pltpu.sync_copy(sp, spv)
    for c in range(NCH):
        off = base + c * CHUNK
        pltpu.sync_copy(relation.at[pl.ds(off, CHUNK)], ir[c])
        for k in range(CHUNK // L):
            sl = pl.ds(k * L, L)
            gsl = pl.ds(c * CHUNK + k * L, L)
            ih[c][sl] = jnp.minimum(ihr[gsl], NEM - 1)
            it[c][sl] = jnp.minimum(itr[gsl], NEM - 1)

    for c in range(NCH):
        sl = pl.ds(c * CHUNK, CHUNK)
        pltpu.async_copy(r_mod.at[ir[c]], rm.at[sl], sem)
        pltpu.async_copy(r_ph.at[ir[c]], rp.at[sl], sem)

    def issue_body(f, carry):
        for c in range(NCH):
            dst = pl.ds(f * BPW + c * CHUNK, CHUNK)
            pltpu.async_copy(emt.at[f].at[ih[c]], hm.at[dst], sem)
            pltpu.async_copy(ept.at[f].at[ih[c]], hp.at[dst], sem)
            pltpu.async_copy(emt.at[f].at[it[c]], tm.at[dst], sem)
            pltpu.async_copy(ept.at[f].at[it[c]], tp.at[dst], sem)
        return carry

    lax.fori_loop(0, D, issue_body, 0)

    for buf in (hm, hp, tm, tp):
        pltpu.make_async_copy(out.at[pl.ds(0, B)], buf, sem).wait()
    for buf in (rm, rp):
        pltpu.make_async_copy(r_mod.at[pl.ds(0, BPW), :], buf, sem).wait()

    row_iota = lax.iota(jnp.int32, L)

    def group_body(g, carry):
        g16 = g * L
        rows = row_iota + g16
        hv = ihr[pl.ds(g16, L)]
        tv = itr[pl.ds(g16, L)]
        hmask = hv >= NEM
        tmask = tv >= NEM
        hs = jnp.maximum(hv, NEM) - NEM
        ts = jnp.maximum(tv, NEM) - NEM
        acc_m = None
        acc_p = None
        for f in range(D):
            sl = pl.ds(f * BPW + g16, L)
            fcol = jnp.full((L,), f, jnp.int32)
            rmv = plsc.load_gather(rm, [rows, fcol])
            rpv = plsc.load_gather(rp, [rows, fcol])
            hmv = jnp.where(hmask, plsc.load_gather(smv, [hs + f * NET]), hm[sl])
            tmv = jnp.where(tmask, plsc.load_gather(smv, [ts + f * NET]), tm[sl])
            hpv = jnp.where(hmask, plsc.load_gather(spv, [hs + f * NET]), hp[sl])
            tpv = jnp.where(tmask, plsc.load_gather(spv, [ts + f * NET]), tp[sl])
            d = hmv * rmv - tmv
            sq = d * d
            acc_m = sq if acc_m is None else acc_m + sq
            s = _abs_sin_half(hpv + rpv - tpv)
            acc_p = s if acc_p is None else acc_p + s
        score = MODULUS_WEIGHT * _newton_sqrt(acc_m) + PHASE_WEIGHT * acc_p
        out_v[pl.ds(g16, L)] = score
        return carry

    lax.fori_loop(0, BPW // L, group_body, 0)

    pltpu.sync_copy(out_v, out.at[pl.ds(base, BPW)])


NE = 1000000
NEM = 999936
NET = NE - NEM


def _untile_body(emt, ept, emo, epo, sem):
    cps = []
    for src, dst in ((emt, emo), (ept, epo)):
        for f in range(D):
            cps.append(pltpu.make_async_copy(
                src.at[f, pl.ds(0, NEM)],
                dst.at[pl.ds(f * NEM, NEM)], sem))
    for cp in cps:
        cp.start()
    for cp in cps:
        cp.wait()


@jax.jit
def _untile(emt, ept):
    return pl.pallas_call(
        _untile_body,
        out_shape=[jax.ShapeDtypeStruct((D * NEM,), jnp.float32)] * 2,
        in_specs=[pl.BlockSpec(memory_space=pl.ANY)] * 2,
        out_specs=[pl.BlockSpec(memory_space=pl.ANY)] * 2,
        scratch_shapes=[pltpu.SemaphoreType.DMA],
    )(emt, ept)


@jax.jit
def _transee_score(head, relation, tail, emt, ept, r_mod, r_ph, sm, sp):
    mesh = plsc.VectorSubcoreMesh(core_axis_name="c", subcore_axis_name="s")
    scratch = (
        [pltpu.VMEM((CHUNK,), jnp.int32)] * (3 * NCH)
        + [pltpu.VMEM((B,), jnp.float32)] * 4
        + [pltpu.VMEM((BPW, D), jnp.float32)] * 2
        + [pltpu.VMEM((BPW,), jnp.int32)] * 2
        + [pltpu.VMEM((D * NET,), jnp.float32)] * 2
        + [pltpu.VMEM((BPW,), jnp.float32)]
        + [pltpu.SemaphoreType.DMA]
    )
    return pl.kernel(
        _sc_body,
        out_type=jax.ShapeDtypeStruct((B,), jnp.float32),
        mesh=mesh,
        scratch_types=scratch,
        compiler_params=pltpu.CompilerParams(needs_layout_passes=False,
                                             use_tc_tiling_on_sc=False),
    )(head, relation, tail, emt, ept, r_mod, r_ph, sm, sp)


def kernel(head, relation, tail, entity_modulus, entity_phase,
           relation_modulus, relation_phase):
    emo, epo = _untile(entity_modulus.T, entity_phase.T)
    side_m = entity_modulus[NEM:].T.reshape(-1)
    side_p = entity_phase[NEM:].T.reshape(-1)
    return _transee_score(head, relation, tail,
                          emo.reshape(D, NEM), epo.reshape(D, NEM),
                          relation_modulus, relation_phase, side_m, side_p)


# speedup vs baseline: 8.5785x; 8.5785x over previous
import functools

import jax
import jax.numpy as jnp
from jax import lax
from jax.experimental import pallas as pl
from jax.experimental.pallas import tpu as pltpu
from jax.experimental.pallas import tpu_sc as plsc

B = 16384
D = 32
NC, NS, L = 2, 16, 16
NW = NC * NS
BPW = B // NW
CHUNK = 128
NCH = BPW // CHUNK

MODULUS_WEIGHT = 4.0
PHASE_WEIGHT = 1.0

_INV_2PI = float(0.15915494309189535)
_MAGIC = float(12582912.0)
_PI_HI = float(3.1415927410125732)
_PI_LO = float(-8.742277657347586e-08)
_C3 = float(-1.0 / 6.0)
_C5 = float(1.0 / 120.0)
_C7 = float(-1.0 / 5040.0)
_C9 = float(1.0 / 362880.0)
_C11 = float(-1.0 / 39916800.0)


def _abs_sin_half(p):
    z = p * _INV_2PI
    k = (z + _MAGIC) - _MAGIC
    y = p * 0.5 - k * _PI_HI
    y = y - k * _PI_LO
    y2 = y * y
    t = _C9 + y2 * _C11
    t = _C7 + y2 * t
    t = _C5 + y2 * t
    t = _C3 + y2 * t
    t = 1.0 + y2 * t
    return jnp.abs(y * t)


def _newton_sqrt(x):
    i = lax.bitcast_convert_type(x, jnp.int32)
    i = jnp.int32(0x1FBD1DF5) + lax.shift_right_arithmetic(i, 1)
    y = lax.bitcast_convert_type(i, jnp.float32)
    for _ in range(3):
        y = 0.5 * (y + x / y)
    return y


def _sc_body(head, relation, tail, e_mod, e_ph, r_mod, r_ph, out, *scr):
    idx_refs = scr[:3 * NCH]
    hm, hp, tm, tp, rm, rp = scr[3 * NCH:3 * NCH + 6]
    acc_m, acc_p = scr[3 * NCH + 6:3 * NCH + 8]
    out_v = scr[3 * NCH + 8]
    sem = scr[3 * NCH + 9]

    wid = lax.axis_index("s") * NC + lax.axis_index("c")
    base = wid * BPW

    for c in range(NCH):
        off = base + c * CHUNK
        pltpu.sync_copy(head.at[pl.ds(off, CHUNK)], idx_refs[c])
        pltpu.sync_copy(tail.at[pl.ds(off, CHUNK)], idx_refs[NCH + c])
        pltpu.sync_copy(relation.at[pl.ds(off, CHUNK)], idx_refs[2 * NCH + c])

    copies = []
    for c in range(NCH):
        sl = pl.ds(c * CHUNK, CHUNK)
        ih, it, ir = idx_refs[c], idx_refs[NCH + c], idx_refs[2 * NCH + c]
        copies.append(pltpu.async_copy(e_mod.at[ih], hm.at[sl], sem))
        copies.append(pltpu.async_copy(e_ph.at[ih], hp.at[sl], sem))
        copies.append(pltpu.async_copy(e_mod.at[it], tm.at[sl], sem))
        copies.append(pltpu.async_copy(e_ph.at[it], tp.at[sl], sem))
        copies.append(pltpu.async_copy(r_mod.at[ir], rm.at[sl], sem))
        copies.append(pltpu.async_copy(r_ph.at[ir], rp.at[sl], sem))
    for cp in copies:
        cp.wait()

    row_iota = lax.iota(jnp.int32, L)

    def group_body(g, carry):
        e0 = g * L
        for j in range(L):
            e = e0 + j
            smod = None
            sph = None
            for h in range(2):
                sl = pl.ds(h * L, L)
                d = hm[e, sl] * rm[e, sl] - tm[e, sl]
                sq = d * d
                smod = sq if smod is None else smod + sq
                s = _abs_sin_half(hp[e, sl] + rp[e, sl] - tp[e, sl])
                sph = s if sph is None else sph + s
            acc_m[pl.ds(j * L, L)] = smod
            acc_p[pl.ds(j * L, L)] = sph
        msum = None
        psum = None
        for l in range(L):
            col = row_iota * L + l
            cm = plsc.load_gather(acc_m, [col])
            cp_ = plsc.load_gather(acc_p, [col])
            msum = cm if msum is None else msum + cm
            psum = cp_ if psum is None else psum + cp_
        score = MODULUS_WEIGHT * _newton_sqrt(msum) + PHASE_WEIGHT * psum
        out_v[pl.ds(e0, L)] = score
        return carry

    lax.fori_loop(0, BPW // L, group_body, 0)

    pltpu.sync_copy(out_v, out.at[pl.ds(base, BPW)])


@jax.jit
def _transee_score(head, relation, tail, e_mod, e_ph, r_mod, r_ph):
    mesh = plsc.VectorSubcoreMesh(core_axis_name="c", subcore_axis_name="s")
    scratch = (
        [pltpu.VMEM((CHUNK,), jnp.int32)] * (3 * NCH)
        + [pltpu.VMEM((BPW, D), jnp.float32)] * 6
        + [pltpu.VMEM((L * L,), jnp.float32)] * 2
        + [pltpu.VMEM((BPW,), jnp.float32)]
        + [pltpu.SemaphoreType.DMA]
    )
    return pl.kernel(
        _sc_body,
        out_type=jax.ShapeDtypeStruct((B,), jnp.float32),
        mesh=mesh,
        scratch_types=scratch,
        compiler_params=pltpu.CompilerParams(needs_layout_passes=False,
                                             use_tc_tiling_on_sc=False),
    )(head, relation, tail, e_mod, e_ph, r_mod, r_ph)


def kernel(head, relation, tail, entity_modulus, entity_phase,
           relation_modulus, relation_phase):
    return _transee_score(head, relation, tail, entity_modulus, entity_phase,
                          relation_modulus, relation_phase)
